# baseline (speedup 1.0000x reference)
import numpy as np
import jax
import jax.numpy as jnp
from jax import lax
from jax.experimental import pallas as pl
from jax.experimental.pallas import tpu as pltpu

ALPHA = 1.702
LIMIT = 7.0
FT = 512
TILE = 256


def _nib2val(n):
    m = n & 7
    mag = jnp.where(
        m == 0, 0.0,
        jnp.where(m == 1, 0.5,
        jnp.where(m == 2, 1.0,
        jnp.where(m == 3, 1.5,
        jnp.where(m == 4, 2.0,
        jnp.where(m == 5, 3.0,
        jnp.where(m == 6, 4.0, 6.0)))))))
    return jnp.where(n >= 8, -mag, mag)


def _dot_nt(a, b):
    return lax.dot_general(a, b, (((1,), (1,)), ((), ())),
                           preferred_element_type=jnp.float32)


def _router_kernel(x_ref, rw_ref, rb_ref, i1_ref, i2_ref, w1_ref, w2_ref):
    x = x_ref[...]
    logits = _dot_nt(x, rw_ref[...]) + rb_ref[...]
    ei = lax.broadcasted_iota(jnp.int32, logits.shape, 1)
    m1 = jnp.max(logits, axis=1, keepdims=True)
    is1 = logits == m1
    masked = jnp.where(is1, -jnp.inf, logits)
    m2 = jnp.max(masked, axis=1, keepdims=True)
    is2 = masked == m2
    p1 = 1.0 / (1.0 + jnp.exp(m2 - m1))
    i1_ref[...] = jnp.sum(jnp.where(is1, ei, 0), axis=1, keepdims=True)
    i2_ref[...] = jnp.sum(jnp.where(is2, ei, 0), axis=1, keepdims=True)
    w1_ref[...] = p1
    w2_ref[...] = 1.0 - p1


def _moe_kernel(te_ref, st_ref, tokc_ref, tokr_ref, wc_ref,
                xe_ref, xo_ref, gug_ref, guu_ref, gugs_ref, guus_ref,
                gb_ref, ub_ref, dnb_ref, dns_ref, dnbias_ref, out_ref,
                wgl_ref, wgh_ref, wul_ref, wuh_ref, wdl_ref, wdh_ref):
    i = pl.program_id(0)
    nt = pl.num_programs(0) - 1
    T = xe_ref.shape[0]
    FF = gug_ref.shape[1]
    nsc = gugs_ref.shape[2]
    ndc = dnb_ref.shape[2]

    cur = jnp.minimum(i, nt - 1)
    prev = jnp.maximum(i - 1, 0)
    changed = jnp.logical_and(
        i < nt, jnp.logical_or(i == 0, te_ref[cur] != te_ref[prev]))
    st_deq = st_ref[cur]
    st_cmp = st_ref[prev]

    @pl.when(changed)
    def _dequant():
        ci = lax.broadcasted_iota(jnp.int32, (nsc, 16 * nsc), 1) // 16
        bi = lax.broadcasted_iota(jnp.int32, (nsc, 16 * nsc), 0)
        sel16 = (ci == bi).astype(jnp.bfloat16)
        gsc = jnp.dot(gugs_ref[0].astype(jnp.bfloat16), sel16,
                      preferred_element_type=jnp.float32)
        usc = jnp.dot(guus_ref[0].astype(jnp.bfloat16), sel16,
                      preferred_element_type=jnp.float32)
        gbytes = gug_ref[0].astype(jnp.int32)
        ubytes = guu_ref[0].astype(jnp.int32)
        wgl_ref[st_deq] = (_nib2val(gbytes & 15) * gsc).astype(jnp.bfloat16)
        wgh_ref[st_deq] = (_nib2val(gbytes >> 4) * gsc).astype(jnp.bfloat16)
        wul_ref[st_deq] = (_nib2val(ubytes & 15) * usc).astype(jnp.bfloat16)
        wuh_ref[st_deq] = (_nib2val(ubytes >> 4) * usc).astype(jnp.bfloat16)

        di = lax.broadcasted_iota(jnp.int32, (dns_ref.shape[2], ndc), 1) // 16
        dbi = lax.broadcasted_iota(jnp.int32, (dns_ref.shape[2], ndc), 0)
        seld = (di == dbi).astype(jnp.bfloat16)
        dsc = jnp.dot(dns_ref[0].astype(jnp.bfloat16), seld,
                      preferred_element_type=jnp.float32)
        dbytes = dnb_ref[0].astype(jnp.int32)
        wdl_ref[st_deq] = (_nib2val(dbytes & 15) * dsc).astype(jnp.bfloat16)
        wdh_ref[st_deq] = (_nib2val(dbytes >> 4) * dsc).astype(jnp.bfloat16)

    @pl.when(i > 0)
    def _compute():
        tok_col = tokc_ref[0]
        oh = (lax.broadcasted_iota(jnp.int32, (TILE, T), 1)
              == tok_col).astype(jnp.bfloat16)
        xg_e = jnp.dot(oh, xe_ref[...],
                       preferred_element_type=jnp.float32).astype(jnp.bfloat16)
        xg_o = jnp.dot(oh, xo_ref[...],
                       preferred_element_type=jnp.float32).astype(jnp.bfloat16)

        wgl = wgl_ref[st_cmp]
        wgh = wgh_ref[st_cmp]
        wul = wul_ref[st_cmp]
        wuh = wuh_ref[st_cmp]
        gate = _dot_nt(xg_e, wgl) + _dot_nt(xg_o, wgh) + gb_ref[0]
        up = _dot_nt(xg_e, wul) + _dot_nt(xg_o, wuh) + ub_ref[0]

        gate = jnp.minimum(gate, LIMIT)
        up = jnp.clip(up, -LIMIT, LIMIT)
        glu = gate * (1.0 / (1.0 + jnp.exp(-ALPHA * gate)))
        act = ((up + 1.0) * glu).astype(jnp.bfloat16)

        wdl = wdl_ref[st_cmp]
        wdh = wdh_ref[st_cmp]
        down = jnp.zeros((TILE, wdl_ref.shape[1]), jnp.float32)
        half = FT // 2
        for j in range(FF // FT):
            a_e = act[:, j * FT:j * FT + half]
            a_o = act[:, j * FT + half:(j + 1) * FT]
            down += _dot_nt(a_e, wdl[:, j * half:(j + 1) * half])
            down += _dot_nt(a_o, wdh[:, j * half:(j + 1) * half])

        down_w = ((down + dnbias_ref[0]) * wc_ref[0]).astype(jnp.bfloat16)

        tok_row = tokr_ref[0]
        oht = (lax.broadcasted_iota(jnp.int32, (T, TILE), 0)
               == tok_row).astype(jnp.bfloat16)
        contrib = jnp.dot(oht, down_w, preferred_element_type=jnp.float32)

        @pl.when(i == 1)
        def _():
            out_ref[...] = contrib

        @pl.when(i > 1)
        def _():
            out_ref[...] += contrib


@jax.jit
def kernel(x, router_w, router_b, gu_blocks, gu_scales, gu_bias, dn_blocks,
           dn_scales, dn_bias):
    Bb, Tt, H = x.shape
    E, FF2 = gu_bias.shape
    FF = FF2 // 2
    T = Bb * Tt
    NPAD = 2 * T + E * TILE
    NT = NPAD // TILE

    xf = x.reshape(T, H)
    xe = xf[:, 0::2].astype(jnp.bfloat16)
    xo = xf[:, 1::2].astype(jnp.bfloat16)

    idx = np.arange(FF)
    within = idx % FT
    base = (idx // FT) * FT
    P = base + np.where(within < FT // 2, 2 * within,
                        2 * (within - FT // 2) + 1)

    gu_b_flat = gu_blocks.reshape(E, 2 * FF, H // 2)
    gug = gu_b_flat[:, 2 * P, :]
    guu = gu_b_flat[:, 2 * P + 1, :]
    gu_s = jnp.exp2(gu_scales.astype(jnp.float32) - 127.0)
    gugs = gu_s[:, 2 * P, :]
    guus = gu_s[:, 2 * P + 1, :]
    gb = gu_bias[:, 2 * P].reshape(E, 1, FF)
    ub = gu_bias[:, 2 * P + 1].reshape(E, 1, FF)

    dnb = dn_blocks.reshape(E, H, FF // 2)
    dns = jnp.exp2(dn_scales.astype(jnp.float32) - 127.0)
    dnbias = dn_bias.reshape(E, 1, H)

    i1, i2, w1, w2 = pl.pallas_call(
        _router_kernel,
        out_shape=[jax.ShapeDtypeStruct((T, 1), jnp.int32),
                   jax.ShapeDtypeStruct((T, 1), jnp.int32),
                   jax.ShapeDtypeStruct((T, 1), jnp.float32),
                   jax.ShapeDtypeStruct((T, 1), jnp.float32)],
    )(xf, router_w, router_b.reshape(1, E))

    ids = jnp.concatenate([i1[:, 0], i2[:, 0]])
    tws = jnp.concatenate([w1[:, 0], w2[:, 0]])
    toks = jnp.concatenate([jnp.arange(T, dtype=jnp.int32)] * 2)
    order = jnp.argsort(ids)
    ids_s = ids[order]
    toks_s = toks[order]
    tws_s = tws[order]
    counts = jnp.sum((ids[None, :] == jnp.arange(E)[:, None]), axis=1)
    pc = ((counts + TILE - 1) // TILE) * TILE
    ps_full = jnp.concatenate([jnp.zeros((1,), pc.dtype), jnp.cumsum(pc)])
    starts = jnp.concatenate([jnp.zeros((1,), counts.dtype),
                              jnp.cumsum(counts)])
    rank = jnp.arange(2 * T) - starts[ids_s]
    pos = ps_full[ids_s] + rank
    row_token = jnp.zeros((NPAD,), jnp.int32).at[pos].set(toks_s)
    row_w = jnp.zeros((NPAD,), jnp.float32).at[pos].set(tws_s)
    tile_expert = jnp.clip(
        jnp.sum(jnp.arange(NT)[:, None] * TILE >= ps_full[None, 1:],
                axis=1), 0, E - 1).astype(jnp.int32)

    tokc = row_token.reshape(NT, TILE, 1)
    tokr = row_token.reshape(NT, 1, TILE)
    wc = row_w.reshape(NT, TILE, 1)

    run_id = jnp.cumsum(jnp.concatenate(
        [jnp.zeros((1,), jnp.int32),
         (tile_expert[1:] != tile_expert[:-1]).astype(jnp.int32)]))
    set_idx = (run_id % 2).astype(jnp.int32)

    def _cur(i):
        return jnp.minimum(i, NT - 1)

    def _prev(i):
        return jnp.maximum(i - 1, 0)

    grid_spec = pltpu.PrefetchScalarGridSpec(
        num_scalar_prefetch=2,
        grid=(NT + 1,),
        in_specs=[
            pl.BlockSpec((1, TILE, 1), lambda i, te, st: (_prev(i), 0, 0)),
            pl.BlockSpec((1, 1, TILE), lambda i, te, st: (_prev(i), 0, 0)),
            pl.BlockSpec((1, TILE, 1), lambda i, te, st: (_prev(i), 0, 0)),
            pl.BlockSpec((T, H // 2), lambda i, te, st: (0, 0)),
            pl.BlockSpec((T, H // 2), lambda i, te, st: (0, 0)),
            pl.BlockSpec((1, FF, H // 2),
                         lambda i, te, st: (te[_cur(i)], 0, 0)),
            pl.BlockSpec((1, FF, H // 2),
                         lambda i, te, st: (te[_cur(i)], 0, 0)),
            pl.BlockSpec((1, FF, H // 32),
                         lambda i, te, st: (te[_cur(i)], 0, 0)),
            pl.BlockSpec((1, FF, H // 32),
                         lambda i, te, st: (te[_cur(i)], 0, 0)),
            pl.BlockSpec((1, 1, FF),
                         lambda i, te, st: (te[_prev(i)], 0, 0)),
            pl.BlockSpec((1, 1, FF),
                         lambda i, te, st: (te[_prev(i)], 0, 0)),
            pl.BlockSpec((1, H, FF // 2),
                         lambda i, te, st: (te[_cur(i)], 0, 0)),
            pl.BlockSpec((1, H, FF // 32),
                         lambda i, te, st: (te[_cur(i)], 0, 0)),
            pl.BlockSpec((1, 1, H),
                         lambda i, te, st: (te[_prev(i)], 0, 0)),
        ],
        out_specs=pl.BlockSpec((T, H), lambda i, te, st: (0, 0)),
        scratch_shapes=[
            pltpu.VMEM((2, FF, H // 2), jnp.bfloat16),
            pltpu.VMEM((2, FF, H // 2), jnp.bfloat16),
            pltpu.VMEM((2, FF, H // 2), jnp.bfloat16),
            pltpu.VMEM((2, FF, H // 2), jnp.bfloat16),
            pltpu.VMEM((2, H, FF // 2), jnp.bfloat16),
            pltpu.VMEM((2, H, FF // 2), jnp.bfloat16),
        ],
    )

    out = pl.pallas_call(
        _moe_kernel,
        grid_spec=grid_spec,
        out_shape=jax.ShapeDtypeStruct((T, H), jnp.float32),
    )(tile_expert, set_idx, tokc, tokr, wc, xe, xo, gug, guu, gugs, guus,
      gb, ub, dnb, dns, dnbias)

    return out.reshape(Bb, Tt, H)


# speedup vs baseline: 1.0405x; 1.0405x over previous
import numpy as np
import jax
import jax.numpy as jnp
from jax import lax
from jax.experimental import pallas as pl
from jax.experimental.pallas import tpu as pltpu

ALPHA = 1.702
LIMIT = 7.0
FT = 512
TILE = 256


def _nib2val(n):
    m = n & 7
    hi = ((m & 3) + 2 + (m == 7)).astype(jnp.float32)
    mag = jnp.where(m < 4, m.astype(jnp.float32) * 0.5, hi)
    return jnp.where(n >= 8, -mag, mag)


def _dot_nt(a, b):
    return lax.dot_general(a, b, (((1,), (1,)), ((), ())),
                           preferred_element_type=jnp.float32)


def _router_kernel(x_ref, rw_ref, rb_ref, i1_ref, i2_ref, w1_ref, w2_ref):
    x = x_ref[...]
    logits = _dot_nt(x, rw_ref[...]) + rb_ref[...]
    ei = lax.broadcasted_iota(jnp.int32, logits.shape, 1)
    m1 = jnp.max(logits, axis=1, keepdims=True)
    is1 = logits == m1
    masked = jnp.where(is1, -jnp.inf, logits)
    m2 = jnp.max(masked, axis=1, keepdims=True)
    is2 = masked == m2
    p1 = 1.0 / (1.0 + jnp.exp(m2 - m1))
    i1_ref[...] = jnp.sum(jnp.where(is1, ei, 0), axis=1, keepdims=True)
    i2_ref[...] = jnp.sum(jnp.where(is2, ei, 0), axis=1, keepdims=True)
    w1_ref[...] = p1
    w2_ref[...] = 1.0 - p1


def _moe_kernel(te_ref, st_ref, tokc_ref, tokr_ref, wc_ref,
                xe_ref, xo_ref, gug_ref, guu_ref, gugs_ref, guus_ref,
                gb_ref, ub_ref, dnb_ref, dns_ref, dnbias_ref, out_ref,
                wgl_ref, wgh_ref, wul_ref, wuh_ref, wdl_ref, wdh_ref):
    i = pl.program_id(0)
    nt = pl.num_programs(0) - 1
    T = xe_ref.shape[0]
    FF = gug_ref.shape[1]
    nsc = gugs_ref.shape[2]
    ndc = dnb_ref.shape[2]

    cur = jnp.minimum(i, nt - 1)
    prev = jnp.maximum(i - 1, 0)
    changed = jnp.logical_and(
        i < nt, jnp.logical_or(i == 0, te_ref[cur] != te_ref[prev]))
    st_deq = st_ref[cur]
    st_cmp = st_ref[prev]

    @pl.when(changed)
    def _dequant():
        ci = lax.broadcasted_iota(jnp.int32, (nsc, 16 * nsc), 1) // 16
        bi = lax.broadcasted_iota(jnp.int32, (nsc, 16 * nsc), 0)
        sel16 = (ci == bi).astype(jnp.bfloat16)
        gsc = jnp.dot(gugs_ref[0].astype(jnp.bfloat16), sel16,
                      preferred_element_type=jnp.float32)
        usc = jnp.dot(guus_ref[0].astype(jnp.bfloat16), sel16,
                      preferred_element_type=jnp.float32)
        gbytes = gug_ref[0].astype(jnp.int32)
        ubytes = guu_ref[0].astype(jnp.int32)
        wgl_ref[st_deq] = (_nib2val(gbytes & 15) * gsc).astype(jnp.bfloat16)
        wgh_ref[st_deq] = (_nib2val(gbytes >> 4) * gsc).astype(jnp.bfloat16)
        wul_ref[st_deq] = (_nib2val(ubytes & 15) * usc).astype(jnp.bfloat16)
        wuh_ref[st_deq] = (_nib2val(ubytes >> 4) * usc).astype(jnp.bfloat16)

        di = lax.broadcasted_iota(jnp.int32, (dns_ref.shape[2], ndc), 1) // 16
        dbi = lax.broadcasted_iota(jnp.int32, (dns_ref.shape[2], ndc), 0)
        seld = (di == dbi).astype(jnp.bfloat16)
        dsc = jnp.dot(dns_ref[0].astype(jnp.bfloat16), seld,
                      preferred_element_type=jnp.float32)
        dbytes = dnb_ref[0].astype(jnp.int32)
        wdl_ref[st_deq] = (_nib2val(dbytes & 15) * dsc).astype(jnp.bfloat16)
        wdh_ref[st_deq] = (_nib2val(dbytes >> 4) * dsc).astype(jnp.bfloat16)

    @pl.when(i > 0)
    def _compute():
        tok_col = tokc_ref[0]
        oh = (lax.broadcasted_iota(jnp.int32, (TILE, T), 1)
              == tok_col).astype(jnp.bfloat16)
        xg_e = jnp.dot(oh, xe_ref[...],
                       preferred_element_type=jnp.float32).astype(jnp.bfloat16)
        xg_o = jnp.dot(oh, xo_ref[...],
                       preferred_element_type=jnp.float32).astype(jnp.bfloat16)

        wgl = wgl_ref[st_cmp]
        wgh = wgh_ref[st_cmp]
        wul = wul_ref[st_cmp]
        wuh = wuh_ref[st_cmp]
        gate = _dot_nt(xg_e, wgl) + _dot_nt(xg_o, wgh) + gb_ref[0]
        up = _dot_nt(xg_e, wul) + _dot_nt(xg_o, wuh) + ub_ref[0]

        gate = jnp.minimum(gate, LIMIT)
        up = jnp.clip(up, -LIMIT, LIMIT)
        glu = gate * (1.0 / (1.0 + jnp.exp(-ALPHA * gate)))
        act = ((up + 1.0) * glu).astype(jnp.bfloat16)

        wdl = wdl_ref[st_cmp]
        wdh = wdh_ref[st_cmp]
        down = jnp.zeros((TILE, wdl_ref.shape[1]), jnp.float32)
        half = FT // 2
        for j in range(FF // FT):
            a_e = act[:, j * FT:j * FT + half]
            a_o = act[:, j * FT + half:(j + 1) * FT]
            down += _dot_nt(a_e, wdl[:, j * half:(j + 1) * half])
            down += _dot_nt(a_o, wdh[:, j * half:(j + 1) * half])

        down_w = ((down + dnbias_ref[0]) * wc_ref[0]).astype(jnp.bfloat16)

        tok_row = tokr_ref[0]
        oht = (lax.broadcasted_iota(jnp.int32, (T, TILE), 0)
               == tok_row).astype(jnp.bfloat16)
        contrib = jnp.dot(oht, down_w, preferred_element_type=jnp.float32)

        @pl.when(i == 1)
        def _():
            out_ref[...] = contrib

        @pl.when(i > 1)
        def _():
            out_ref[...] += contrib


@jax.jit
def kernel(x, router_w, router_b, gu_blocks, gu_scales, gu_bias, dn_blocks,
           dn_scales, dn_bias):
    Bb, Tt, H = x.shape
    E, FF2 = gu_bias.shape
    FF = FF2 // 2
    T = Bb * Tt
    NPAD = 2 * T + E * TILE
    NT = NPAD // TILE

    xf = x.reshape(T, H)
    xe = xf[:, 0::2].astype(jnp.bfloat16)
    xo = xf[:, 1::2].astype(jnp.bfloat16)

    idx = np.arange(FF)
    within = idx % FT
    base = (idx // FT) * FT
    P = base + np.where(within < FT // 2, 2 * within,
                        2 * (within - FT // 2) + 1)

    gu_b_flat = gu_blocks.reshape(E, 2 * FF, H // 2)
    gug = gu_b_flat[:, 2 * P, :]
    guu = gu_b_flat[:, 2 * P + 1, :]
    gu_s = jnp.exp2(gu_scales.astype(jnp.float32) - 127.0)
    gugs = gu_s[:, 2 * P, :]
    guus = gu_s[:, 2 * P + 1, :]
    gb = gu_bias[:, 2 * P].reshape(E, 1, FF)
    ub = gu_bias[:, 2 * P + 1].reshape(E, 1, FF)

    dnb = dn_blocks.reshape(E, H, FF // 2)
    dns = jnp.exp2(dn_scales.astype(jnp.float32) - 127.0)
    dnbias = dn_bias.reshape(E, 1, H)

    i1, i2, w1, w2 = pl.pallas_call(
        _router_kernel,
        out_shape=[jax.ShapeDtypeStruct((T, 1), jnp.int32),
                   jax.ShapeDtypeStruct((T, 1), jnp.int32),
                   jax.ShapeDtypeStruct((T, 1), jnp.float32),
                   jax.ShapeDtypeStruct((T, 1), jnp.float32)],
    )(xf, router_w, router_b.reshape(1, E))

    ids = jnp.concatenate([i1[:, 0], i2[:, 0]])
    tws = jnp.concatenate([w1[:, 0], w2[:, 0]])
    toks = jnp.concatenate([jnp.arange(T, dtype=jnp.int32)] * 2)
    order = jnp.argsort(ids)
    ids_s = ids[order]
    toks_s = toks[order]
    tws_s = tws[order]
    counts = jnp.sum((ids[None, :] == jnp.arange(E)[:, None]), axis=1)
    pc = ((counts + TILE - 1) // TILE) * TILE
    ps_full = jnp.concatenate([jnp.zeros((1,), pc.dtype), jnp.cumsum(pc)])
    starts = jnp.concatenate([jnp.zeros((1,), counts.dtype),
                              jnp.cumsum(counts)])
    rank = jnp.arange(2 * T) - starts[ids_s]
    pos = ps_full[ids_s] + rank
    row_token = jnp.zeros((NPAD,), jnp.int32).at[pos].set(toks_s)
    row_w = jnp.zeros((NPAD,), jnp.float32).at[pos].set(tws_s)
    tile_expert = jnp.clip(
        jnp.sum(jnp.arange(NT)[:, None] * TILE >= ps_full[None, 1:],
                axis=1), 0, E - 1).astype(jnp.int32)

    tokc = row_token.reshape(NT, TILE, 1)
    tokr = row_token.reshape(NT, 1, TILE)
    wc = row_w.reshape(NT, TILE, 1)

    run_id = jnp.cumsum(jnp.concatenate(
        [jnp.zeros((1,), jnp.int32),
         (tile_expert[1:] != tile_expert[:-1]).astype(jnp.int32)]))
    set_idx = (run_id % 2).astype(jnp.int32)

    def _cur(i):
        return jnp.minimum(i, NT - 1)

    def _prev(i):
        return jnp.maximum(i - 1, 0)

    grid_spec = pltpu.PrefetchScalarGridSpec(
        num_scalar_prefetch=2,
        grid=(NT + 1,),
        in_specs=[
            pl.BlockSpec((1, TILE, 1), lambda i, te, st: (_prev(i), 0, 0)),
            pl.BlockSpec((1, 1, TILE), lambda i, te, st: (_prev(i), 0, 0)),
            pl.BlockSpec((1, TILE, 1), lambda i, te, st: (_prev(i), 0, 0)),
            pl.BlockSpec((T, H // 2), lambda i, te, st: (0, 0)),
            pl.BlockSpec((T, H // 2), lambda i, te, st: (0, 0)),
            pl.BlockSpec((1, FF, H // 2),
                         lambda i, te, st: (te[_cur(i)], 0, 0)),
            pl.BlockSpec((1, FF, H // 2),
                         lambda i, te, st: (te[_cur(i)], 0, 0)),
            pl.BlockSpec((1, FF, H // 32),
                         lambda i, te, st: (te[_cur(i)], 0, 0)),
            pl.BlockSpec((1, FF, H // 32),
                         lambda i, te, st: (te[_cur(i)], 0, 0)),
            pl.BlockSpec((1, 1, FF),
                         lambda i, te, st: (te[_prev(i)], 0, 0)),
            pl.BlockSpec((1, 1, FF),
                         lambda i, te, st: (te[_prev(i)], 0, 0)),
            pl.BlockSpec((1, H, FF // 2),
                         lambda i, te, st: (te[_cur(i)], 0, 0)),
            pl.BlockSpec((1, H, FF // 32),
                         lambda i, te, st: (te[_cur(i)], 0, 0)),
            pl.BlockSpec((1, 1, H),
                         lambda i, te, st: (te[_prev(i)], 0, 0)),
        ],
        out_specs=pl.BlockSpec((T, H), lambda i, te, st: (0, 0)),
        scratch_shapes=[
            pltpu.VMEM((2, FF, H // 2), jnp.bfloat16),
            pltpu.VMEM((2, FF, H // 2), jnp.bfloat16),
            pltpu.VMEM((2, FF, H // 2), jnp.bfloat16),
            pltpu.VMEM((2, FF, H // 2), jnp.bfloat16),
            pltpu.VMEM((2, H, FF // 2), jnp.bfloat16),
            pltpu.VMEM((2, H, FF // 2), jnp.bfloat16),
        ],
    )

    out = pl.pallas_call(
        _moe_kernel,
        grid_spec=grid_spec,
        out_shape=jax.ShapeDtypeStruct((T, H), jnp.float32),
    )(tile_expert, set_idx, tokc, tokr, wc, xe, xo, gug, guu, gugs, guus,
      gb, ub, dnb, dns, dnbias)

    return out.reshape(Bb, Tt, H)
